# Initial kernel scaffold; baseline (speedup 1.0000x reference)
import jax
import jax.numpy as jnp
from jax.experimental import pallas as pl


def kernel(inputs):
    raise NotImplementedError("write your pallas kernel here")


# speedup vs baseline: 12.8683x; 12.8683x over previous
import jax
import jax.numpy as jnp
from jax.experimental import pallas as pl
from jax.experimental.pallas import tpu as pltpu

_K = 16
_R = 256


def _knn_body(keys_ref, q_ref, out_ref):
    keys = keys_ref[0]
    q = q_ref[0]
    n = keys.shape[0]
    r = q.shape[0]
    inner = jax.lax.dot_general(
        q, keys, (((1,), (1,)), ((), ())),
        preferred_element_type=jnp.float32)
    qn = jnp.sum(q * q, axis=1, keepdims=True)
    kn = jnp.sum(keys * keys, axis=1)
    d = qn - 2.0 * inner + kn[None, :]
    colid = jax.lax.broadcasted_iota(jnp.int32, (r, n), 1)
    big = jnp.float32(jnp.inf)
    out = jnp.zeros((r, _K), jnp.int32)
    kcol = jax.lax.broadcasted_iota(jnp.int32, (r, _K), 1)
    for k in range(_K):
        m = jnp.min(d, axis=1, keepdims=True)
        c = jnp.where(d == m, colid, jnp.int32(n))
        idx = jnp.min(c, axis=1, keepdims=True)
        out = jnp.where(kcol == k, idx, out)
        d = jnp.where(colid == idx, big, d)
    out_ref[0] = out


def kernel(inputs):
    b, n, d = inputs.shape
    grid = (b, n // _R)
    return pl.pallas_call(
        _knn_body,
        grid=grid,
        in_specs=[
            pl.BlockSpec((1, n, d), lambda bi, ri: (bi, 0, 0)),
            pl.BlockSpec((1, _R, d), lambda bi, ri: (bi, ri, 0)),
        ],
        out_specs=pl.BlockSpec((1, _R, _K), lambda bi, ri: (bi, ri, 0)),
        out_shape=jax.ShapeDtypeStruct((b, n, _K), jnp.int32),
        compiler_params=pltpu.CompilerParams(
            dimension_semantics=("parallel", "arbitrary")),
    )(inputs, inputs)
